# baseline (speedup 1.0000x reference)
import functools

import jax
import jax.numpy as jnp
from jax import lax
from jax.experimental import pallas as pl
from jax.experimental.pallas import tpu as pltpu
from jax.experimental.pallas import tpu_sc as plsc

LANES = 16
CHUNK_BAGS = 4
SPLIT = 104
NBUF = 4


def _sc_pooled_sum(xf, table, batch, seq, dim):
    info = plsc.get_sparse_core_info()
    nc, ns = info.num_cores, info.num_subcores
    nw = nc * ns
    bags_w = batch // nw
    rpc = CHUNK_BAGS * seq
    nchunks_w = bags_w // CHUNK_BAGS
    idx_w = bags_w * seq
    mesh = plsc.VectorSubcoreMesh(core_axis_name="c", subcore_axis_name="s")

    @functools.partial(
        pl.kernel,
        mesh=mesh,
        compiler_params=pltpu.CompilerParams(
            use_tc_tiling_on_sc=False, needs_layout_passes=False
        ),
        out_type=jax.ShapeDtypeStruct((batch, dim), jnp.float32),
        scratch_types=[
            pltpu.VMEM((idx_w,), jnp.int32),
            pltpu.VMEM((NBUF, SPLIT, dim // 2), jnp.int32),
            pltpu.VMEM((NBUF, rpc - SPLIT, dim // 2), jnp.int32),
            pltpu.VMEM((bags_w, dim), jnp.float32),
            pltpu.SemaphoreType.DMA,
        ],
    )
    def sc_kernel(x_hbm, table_hbm, out_hbm, idx_v, rows_a, rows_b, out_v, sem):
        wid = lax.axis_index("s") * nc + lax.axis_index("c")
        pltpu.sync_copy(x_hbm.at[pl.ds(wid * idx_w, idx_w)], idx_v)

        def descs(chunk, slot):
            base = chunk * rpc
            da = pltpu.make_async_copy(
                table_hbm.at[idx_v.at[pl.ds(base, SPLIT)]],
                rows_a.at[slot], sem)
            db = pltpu.make_async_copy(
                table_hbm.at[idx_v.at[pl.ds(base + SPLIT, rpc - SPLIT)]],
                rows_b.at[slot], sem)
            return da, db

        def gather(chunk, slot):
            da, db = descs(chunk, slot)
            da.start()
            db.start()

        def wait_slot(chunk, slot):
            da, db = descs(chunk, slot)
            da.wait()
            db.wait()

        for b in range(NBUF):
            gather(b, b)

        def body(j, carry):
            slot = lax.rem(j, NBUF)
            wait_slot(j, slot)
            ra = rows_a.at[slot]
            rb = rows_b.at[slot]
            for bag in range(CHUNK_BAGS):

                def ld(gr):
                    if gr < SPLIT:
                        w = ra[gr]
                    else:
                        w = rb[gr - SPLIT]
                    e = plsc.bitcast(w << 16, jnp.float32)
                    o = plsc.bitcast(w & jnp.int32(-65536), jnp.float32)
                    return e, o

                base = bag * seq
                s0e, s0o = ld(base)
                s1e, s1o = ld(base + 1)
                for l in range(2, seq, 2):
                    e, o = ld(base + l)
                    s0e, s0o = s0e + e, s0o + o
                    e, o = ld(base + l + 1)
                    s1e, s1o = s1e + e, s1o + o
                row = j * CHUNK_BAGS + bag
                out_v[row, pl.ds(0, LANES)] = s0e + s1e
                out_v[row, pl.ds(LANES, LANES)] = s0o + s1o
            nj = j + NBUF

            @pl.when(nj < nchunks_w)
            def _():
                gather(nj, slot)

            return carry

        lax.fori_loop(0, nchunks_w, body, 0)
        pltpu.sync_copy(out_v, out_hbm.at[pl.ds(wid * bags_w, bags_w)])

    return sc_kernel(xf, table)


def _tc_head(pooled_sum, gamma, beta, w, bias, seq):
    batch, dim = pooled_sum.shape
    out_dim = w.shape[0]
    blk = 2048
    inv_n = 1.0 / seq

    def body(ps_ref, g_ref, bt_ref, w_ref, bias_ref, o_ref):
        ps = ps_ref[...] * inv_n
        mu = jnp.mean(ps, axis=1, keepdims=True)
        xc = ps - mu
        var = jnp.mean(xc * xc, axis=1, keepdims=True)
        h = xc * lax.rsqrt(var + 1e-5) * g_ref[...] + bt_ref[...]
        h = jnp.maximum(h, 0.0)
        o_ref[...] = (
            lax.dot_general(
                h, w_ref[...], (((1,), (1,)), ((), ())),
                preferred_element_type=jnp.float32,
            )
            + bias_ref[...]
        )

    return pl.pallas_call(
        body,
        grid=(batch // blk,),
        in_specs=[
            pl.BlockSpec((blk, dim), lambda i: (i, 0)),
            pl.BlockSpec((1, dim), lambda i: (0, 0)),
            pl.BlockSpec((1, dim), lambda i: (0, 0)),
            pl.BlockSpec((out_dim, dim), lambda i: (0, 0)),
            pl.BlockSpec((1, out_dim), lambda i: (0, 0)),
        ],
        out_specs=pl.BlockSpec((blk, out_dim), lambda i: (i, 0)),
        out_shape=jax.ShapeDtypeStruct((batch, out_dim), jnp.float32),
    )(
        pooled_sum,
        gamma.reshape(1, dim),
        beta.reshape(1, dim),
        w,
        bias.reshape(1, out_dim),
    )


def kernel(x, table, ln_gamma, ln_beta, W, b):
    batch, seq = x.shape
    dim = table.shape[1]
    xf = x.reshape(-1)
    table_p = lax.bitcast_convert_type(
        table.astype(jnp.bfloat16).reshape(-1, dim // 2, 2), jnp.int32
    )
    pooled_sum = _sc_pooled_sum(xf, table_p, batch, seq, dim)
    perm = jnp.array(
        [2 * k for k in range(dim // 2)] + [2 * k + 1 for k in range(dim // 2)],
        dtype=jnp.int32,
    )
    return _tc_head(
        pooled_sum, ln_gamma[perm], ln_beta[perm], W[:, perm], b, seq
    )


# speedup vs baseline: 1.6800x; 1.6800x over previous
import functools

import jax
import jax.numpy as jnp
from jax import lax
from jax.experimental import pallas as pl
from jax.experimental.pallas import tpu as pltpu
from jax.experimental.pallas import tpu_sc as plsc

LANES = 16
CHUNK_BAGS = 4
SPLIT = 104
NBUF = 4


def _sc_pooled_sum(xf, table, batch, seq, dim):
    info = plsc.get_sparse_core_info()
    nc, ns = info.num_cores, info.num_subcores
    nw = nc * ns
    bags_w = batch // nw
    rpc = CHUNK_BAGS * seq
    nchunks_w = bags_w // CHUNK_BAGS
    idx_w = bags_w * seq
    mesh = plsc.VectorSubcoreMesh(core_axis_name="c", subcore_axis_name="s")

    @functools.partial(
        pl.kernel,
        mesh=mesh,
        compiler_params=pltpu.CompilerParams(
            use_tc_tiling_on_sc=False, needs_layout_passes=False
        ),
        out_type=jax.ShapeDtypeStruct((batch, dim), jnp.float32),
        scratch_types=[
            pltpu.VMEM((idx_w,), jnp.int32),
            pltpu.VMEM((NBUF, SPLIT, dim), jnp.bfloat16),
            pltpu.VMEM((NBUF, rpc - SPLIT, dim), jnp.bfloat16),
            pltpu.VMEM((bags_w, dim), jnp.float32),
            pltpu.SemaphoreType.DMA,
        ],
    )
    def sc_kernel(x_hbm, table_hbm, out_hbm, idx_v, rows_a, rows_b, out_v, sem):
        wid = lax.axis_index("s") * nc + lax.axis_index("c")
        pltpu.sync_copy(x_hbm.at[pl.ds(wid * idx_w, idx_w)], idx_v)

        def descs(chunk, slot):
            base = chunk * rpc
            da = pltpu.make_async_copy(
                table_hbm.at[idx_v.at[pl.ds(base, SPLIT)]],
                rows_a.at[slot], sem)
            db = pltpu.make_async_copy(
                table_hbm.at[idx_v.at[pl.ds(base + SPLIT, rpc - SPLIT)]],
                rows_b.at[slot], sem)
            return da, db

        def gather(chunk, slot):
            da, db = descs(chunk, slot)
            da.start()
            db.start()

        def wait_slot(chunk, slot):
            da, db = descs(chunk, slot)
            da.wait()
            db.wait()

        for b in range(NBUF):
            gather(b, b)

        def body(j, carry):
            slot = lax.rem(j, NBUF)
            wait_slot(j, slot)
            ra = rows_a.at[slot]
            rb = rows_b.at[slot]
            for bag in range(CHUNK_BAGS):

                def ld(gr):
                    if gr < SPLIT:
                        v = ra[gr]
                    else:
                        v = rb[gr - SPLIT]
                    return plsc.unpack(v, format=plsc.PackFormat.INTERLEAVED)

                base = bag * seq
                s0e, s0o = ld(base)
                s1e, s1o = ld(base + 1)
                for l in range(2, seq, 2):
                    e, o = ld(base + l)
                    s0e, s0o = s0e + e, s0o + o
                    e, o = ld(base + l + 1)
                    s1e, s1o = s1e + e, s1o + o
                row = j * CHUNK_BAGS + bag
                out_v[row, pl.ds(0, LANES)] = s0e + s1e
                out_v[row, pl.ds(LANES, LANES)] = s0o + s1o
            nj = j + NBUF

            @pl.when(nj < nchunks_w)
            def _():
                gather(nj, slot)

            return carry

        lax.fori_loop(0, nchunks_w, body, 0)
        pltpu.sync_copy(out_v, out_hbm.at[pl.ds(wid * bags_w, bags_w)])

    return sc_kernel(xf, table)


def _tc_head(pooled_sum, gamma, beta, w, bias, seq):
    batch, dim = pooled_sum.shape
    out_dim = w.shape[0]
    blk = 2048
    inv_n = 1.0 / seq

    def body(ps_ref, g_ref, bt_ref, w_ref, bias_ref, o_ref):
        ps = ps_ref[...] * inv_n
        mu = jnp.mean(ps, axis=1, keepdims=True)
        xc = ps - mu
        var = jnp.mean(xc * xc, axis=1, keepdims=True)
        h = xc * lax.rsqrt(var + 1e-5) * g_ref[...] + bt_ref[...]
        h = jnp.maximum(h, 0.0)
        o_ref[...] = (
            lax.dot_general(
                h, w_ref[...], (((1,), (1,)), ((), ())),
                preferred_element_type=jnp.float32,
            )
            + bias_ref[...]
        )

    return pl.pallas_call(
        body,
        grid=(batch // blk,),
        in_specs=[
            pl.BlockSpec((blk, dim), lambda i: (i, 0)),
            pl.BlockSpec((1, dim), lambda i: (0, 0)),
            pl.BlockSpec((1, dim), lambda i: (0, 0)),
            pl.BlockSpec((out_dim, dim), lambda i: (0, 0)),
            pl.BlockSpec((1, out_dim), lambda i: (0, 0)),
        ],
        out_specs=pl.BlockSpec((blk, out_dim), lambda i: (i, 0)),
        out_shape=jax.ShapeDtypeStruct((batch, out_dim), jnp.float32),
    )(
        pooled_sum,
        gamma.reshape(1, dim),
        beta.reshape(1, dim),
        w,
        bias.reshape(1, out_dim),
    )


def kernel(x, table, ln_gamma, ln_beta, W, b):
    batch, seq = x.shape
    dim = table.shape[1]
    xf = x.reshape(-1)
    pooled_sum = _sc_pooled_sum(xf, table.astype(jnp.bfloat16), batch, seq, dim)
    perm = jnp.array(
        [2 * k for k in range(dim // 2)] + [2 * k + 1 for k in range(dim // 2)],
        dtype=jnp.int32,
    )
    return _tc_head(
        pooled_sum, ln_gamma[perm], ln_beta[perm], W[:, perm], b, seq
    )


# speedup vs baseline: 2.0101x; 1.1965x over previous
import functools

import jax
import jax.numpy as jnp
from jax import lax
from jax.experimental import pallas as pl
from jax.experimental.pallas import tpu as pltpu
from jax.experimental.pallas import tpu_sc as plsc

LANES = 16
CHUNK_BAGS = 4
SPLIT = 104
NBUF = 8


def _sc_pooled_sum(xf, table, batch, seq, dim):
    info = plsc.get_sparse_core_info()
    nc, ns = info.num_cores, info.num_subcores
    nw = nc * ns
    bags_w = batch // nw
    rpc = CHUNK_BAGS * seq
    nchunks_w = bags_w // CHUNK_BAGS
    idx_w = bags_w * seq
    mesh = plsc.VectorSubcoreMesh(core_axis_name="c", subcore_axis_name="s")

    @functools.partial(
        pl.kernel,
        mesh=mesh,
        compiler_params=pltpu.CompilerParams(use_tc_tiling_on_sc=False),
        out_type=jax.ShapeDtypeStruct((batch, dim), jnp.float32),
        scratch_types=[
            pltpu.VMEM((idx_w,), jnp.int32),
            pltpu.VMEM((NBUF, SPLIT, dim), jnp.float32),
            pltpu.VMEM((NBUF, rpc - SPLIT, dim), jnp.float32),
            pltpu.VMEM((bags_w, dim), jnp.float32),
            pltpu.SemaphoreType.DMA,
        ],
    )
    def sc_kernel(x_hbm, table_hbm, out_hbm, idx_v, rows_a, rows_b, out_v, sem):
        wid = lax.axis_index("s") * nc + lax.axis_index("c")
        pltpu.sync_copy(x_hbm.at[pl.ds(wid * idx_w, idx_w)], idx_v)

        def descs(chunk, slot):
            base = chunk * rpc
            da = pltpu.make_async_copy(
                table_hbm.at[idx_v.at[pl.ds(base, SPLIT)]],
                rows_a.at[slot], sem)
            db = pltpu.make_async_copy(
                table_hbm.at[idx_v.at[pl.ds(base + SPLIT, rpc - SPLIT)]],
                rows_b.at[slot], sem)
            return da, db

        def gather(chunk, slot):
            da, db = descs(chunk, slot)
            da.start()
            db.start()

        def wait_slot(chunk, slot):
            da, db = descs(chunk, slot)
            da.wait()
            db.wait()

        for b in range(NBUF):
            gather(b, b)

        def body(j, carry):
            slot = lax.rem(j, NBUF)
            wait_slot(j, slot)
            ra = rows_a.at[slot]
            rb = rows_b.at[slot]
            for bag in range(CHUNK_BAGS):
                for h in range(dim // LANES):
                    sl = pl.ds(h * LANES, LANES)

                    def ld(gr):
                        if gr < SPLIT:
                            return ra[gr, sl]
                        return rb[gr - SPLIT, sl]

                    base = bag * seq
                    s0 = ld(base)
                    s1 = ld(base + 1)
                    for l in range(2, seq, 2):
                        s0 = s0 + ld(base + l)
                        s1 = s1 + ld(base + l + 1)
                    out_v[j * CHUNK_BAGS + bag, sl] = s0 + s1
            nj = j + NBUF

            @pl.when(nj < nchunks_w)
            def _():
                gather(nj, slot)

            return carry

        lax.fori_loop(0, nchunks_w, body, 0)
        pltpu.sync_copy(out_v, out_hbm.at[pl.ds(wid * bags_w, bags_w)])

    return sc_kernel(xf, table)


def _tc_head(pooled_sum, gamma, beta, w, bias, seq):
    batch, dim = pooled_sum.shape
    out_dim = w.shape[0]
    blk = 2048
    inv_n = 1.0 / seq

    def body(ps_ref, g_ref, bt_ref, w_ref, bias_ref, o_ref):
        ps = ps_ref[...] * inv_n
        mu = jnp.mean(ps, axis=1, keepdims=True)
        xc = ps - mu
        var = jnp.mean(xc * xc, axis=1, keepdims=True)
        h = xc * lax.rsqrt(var + 1e-5) * g_ref[...] + bt_ref[...]
        h = jnp.maximum(h, 0.0)
        o_ref[...] = (
            lax.dot_general(
                h, w_ref[...], (((1,), (1,)), ((), ())),
                preferred_element_type=jnp.float32,
            )
            + bias_ref[...]
        )

    return pl.pallas_call(
        body,
        grid=(batch // blk,),
        in_specs=[
            pl.BlockSpec((blk, dim), lambda i: (i, 0)),
            pl.BlockSpec((1, dim), lambda i: (0, 0)),
            pl.BlockSpec((1, dim), lambda i: (0, 0)),
            pl.BlockSpec((out_dim, dim), lambda i: (0, 0)),
            pl.BlockSpec((1, out_dim), lambda i: (0, 0)),
        ],
        out_specs=pl.BlockSpec((blk, out_dim), lambda i: (i, 0)),
        out_shape=jax.ShapeDtypeStruct((batch, out_dim), jnp.float32),
    )(
        pooled_sum,
        gamma.reshape(1, dim),
        beta.reshape(1, dim),
        w,
        bias.reshape(1, out_dim),
    )


def kernel(x, table, ln_gamma, ln_beta, W, b):
    batch, seq = x.shape
    dim = table.shape[1]
    xf = x.reshape(-1)
    pooled_sum = _sc_pooled_sum(xf, table, batch, seq, dim)
    return _tc_head(pooled_sum, ln_gamma, ln_beta, W, b, seq)
